# baseline (speedup 1.0000x reference)
import jax
import jax.numpy as jnp
from jax import lax
from jax.experimental import pallas as pl
from jax.experimental.pallas import tpu as pltpu
from jax.experimental.pallas import tpu_sc as plsc

N = 10000
E = 320000
FN = 128
FE = 16
RBF = 16
H1 = 64
MO = 64
MV = 32
GAMMA = 10.0

D = 128
NC, NS, L = 2, 16, 16
NW = NC * NS
CH = 80
NPT = 632
NPA = NPT * NS
DA = 128
ZB = 8

NSLICE = 5
ES = E // NSLICE

_f32 = jnp.float32


def _prep_body(nf_ref, coord_ref, wsrc_ref, wdst_ref, a_ref, b_ref):
    nf = nf_ref[...]
    blk = nf.shape[0]
    c = coord_ref[...]
    cpad = jnp.concatenate([c, jnp.zeros((blk, 16 - 3), _f32)], axis=1)
    zpad = jnp.zeros((blk, D - H1 - 16), _f32)
    p = jnp.dot(nf, wsrc_ref[...], preferred_element_type=_f32)
    q = jnp.dot(nf, wdst_ref[...], preferred_element_type=_f32)
    a_ref[...] = jnp.concatenate([p, -cpad, zpad], axis=1)
    b_ref[...] = jnp.concatenate([q, cpad, zpad], axis=1)


def _prep(node_feat, coord, wsrc, wdst):
    blk = 1000
    grid = N // blk
    return pl.pallas_call(
        _prep_body,
        grid=(grid,),
        in_specs=[
            pl.BlockSpec((blk, FN), lambda i: (i, 0)),
            pl.BlockSpec((blk, 3), lambda i: (i, 0)),
            pl.BlockSpec((FN, H1), lambda i: (0, 0)),
            pl.BlockSpec((FN, H1), lambda i: (0, 0)),
        ],
        out_specs=[
            pl.BlockSpec((blk, D), lambda i: (i, 0)),
            pl.BlockSpec((blk, D), lambda i: (i, 0)),
        ],
        out_shape=[
            jax.ShapeDtypeStruct((N, D), _f32),
            jax.ShapeDtypeStruct((N, D), _f32),
        ],
    )(node_feat, coord, wsrc, wdst)


def _split_body(eix_ref, src_out, dst_out):
    src_out[...] = eix_ref[0, :]
    dst_out[...] = eix_ref[1, :]


def _split(edge_index):
    return pl.pallas_call(
        _split_body,
        out_shape=[
            jax.ShapeDtypeStruct((E,), jnp.int32),
            jax.ShapeDtypeStruct((E,), jnp.int32),
        ],
    )(edge_index)


def _pipeline(nchunk, fire, finish):
    fire(0, 0)
    npairs = (nchunk - 1) // 2

    def pair(j, carry):
        c0 = 2 * j
        fire(c0 + 1, 1)
        finish(c0, 0)
        fire(c0 + 2, 0)
        finish(c0 + 1, 1)
        return carry

    lax.fori_loop(0, npairs, pair, 0)
    if nchunk % 2 == 1:
        finish(nchunk - 1, 0)
    else:
        last = nchunk - 1
        fire(last, 1)
        finish(last - 1, 0)
        finish(last, 1)


def _make_gather(e_total, offset):
    epw = e_total // NW
    nchunk = epw // CH
    assert epw % CH == 0 and (epw % 8) == 0

    def body(src_hbm, dst_hbm, a_hbm, b_hbm, g_hbm,
             sidx0, didx0, rowsa0, rowsb0,
             sidx1, didx1, rowsa1, rowsb1,
             sema0, semb0, sema1, semb1):
        wid = lax.axis_index("s") * NC + lax.axis_index("c")
        base0 = offset + wid * epw
        sidx = (sidx0, sidx1)
        didx = (didx0, didx1)
        rowsa = (rowsa0, rowsa1)
        rowsb = (rowsb0, rowsb1)
        sema = (sema0, sema1)
        semb = (semb0, semb1)

        def fire(c, bufi):
            base = base0 + c * CH
            pltpu.sync_copy(src_hbm.at[pl.ds(base, CH)], sidx[bufi])
            pltpu.sync_copy(dst_hbm.at[pl.ds(base, CH)], didx[bufi])
            pltpu.async_copy(a_hbm.at[sidx[bufi]], rowsa[bufi], sema[bufi])
            pltpu.async_copy(b_hbm.at[didx[bufi]], rowsb[bufi], semb[bufi])

        def finish(c, bufi):
            base = wid * epw + c * CH
            pltpu.make_async_copy(a_hbm.at[sidx[bufi]], rowsa[bufi],
                                  sema[bufi]).wait()
            pltpu.make_async_copy(b_hbm.at[didx[bufi]], rowsb[bufi],
                                  semb[bufi]).wait()
            ra = rowsa[bufi]
            rb = rowsb[bufi]

            def addrow(r, c2):
                for cc in range(D // L):
                    ra[r, pl.ds(cc * L, L)] = (ra[r, pl.ds(cc * L, L)]
                                               + rb[r, pl.ds(cc * L, L)])
                return c2

            lax.fori_loop(0, CH, addrow, 0)
            pltpu.sync_copy(ra, g_hbm.at[pl.ds(base, CH)])

        _pipeline(nchunk, fire, finish)

    mesh = plsc.VectorSubcoreMesh(core_axis_name="c", subcore_axis_name="s",
                                  num_cores=NC, num_subcores=NS)

    def call(src, dst, a, b):
        return pl.kernel(
            body,
            out_type=jax.ShapeDtypeStruct((e_total, D), _f32),
            mesh=mesh,
            scratch_types=[
                pltpu.VMEM((CH,), jnp.int32),
                pltpu.VMEM((CH,), jnp.int32),
                pltpu.VMEM((CH, D), _f32),
                pltpu.VMEM((CH, D), _f32),
                pltpu.VMEM((CH,), jnp.int32),
                pltpu.VMEM((CH,), jnp.int32),
                pltpu.VMEM((CH, D), _f32),
                pltpu.VMEM((CH, D), _f32),
                pltpu.SemaphoreType.DMA,
                pltpu.SemaphoreType.DMA,
                pltpu.SemaphoreType.DMA,
                pltpu.SemaphoreType.DMA,
            ],
        )(src, dst, a, b)

    return call


def _edge_body(g_ref, ef_ref, ones_ref, wct_ref, wd_ref, w2_ref, wm1_ref,
               wm2b_ref, c64_ref, out_ref):
    g = g_ref[...]
    pre = g[:, :H1]
    gt = g[:, H1:]
    ef = ef_ref[...]
    d2 = jnp.dot(gt * gt, ones_ref[...], preferred_element_type=_f32)
    dist = jnp.sqrt(d2 + 1e-12)
    diff = dist - c64_ref[...]
    rbfx = jnp.exp(-GAMMA * diff * diff)
    z = (pre
         + jnp.dot(rbfx, wct_ref[...], preferred_element_type=_f32)
         + jnp.dot(ef, wd_ref[...], preferred_element_type=_f32))
    h = z * jax.nn.sigmoid(z)
    m = jnp.dot(h, w2_ref[...], preferred_element_type=_f32)
    m = m * jax.nn.sigmoid(m)
    mh = jnp.dot(m, wm1_ref[...], preferred_element_type=_f32)
    mh = mh * jax.nn.sigmoid(mh)
    scalb = jnp.dot(mh, wm2b_ref[...], preferred_element_type=_f32)
    out_ref[...] = jnp.concatenate([m, gt * scalb], axis=1)


def _edge(g, ef8, ones64, wct, wd, w2_msg, w1_mov, wm2b, c64, sl):
    e_total = g.shape[0]
    be = 2560
    grid = e_total // be
    boff = sl * (ES // be)
    return pl.pallas_call(
        _edge_body,
        grid=(grid,),
        in_specs=[
            pl.BlockSpec((be, D), lambda i: (i, 0)),
            pl.BlockSpec((be, D), lambda i, boff=boff: (i + boff, 0)),
            pl.BlockSpec((H1, H1), lambda i: (0, 0)),
            pl.BlockSpec((H1, H1), lambda i: (0, 0)),
            pl.BlockSpec((D, H1), lambda i: (0, 0)),
            pl.BlockSpec((H1, MO), lambda i: (0, 0)),
            pl.BlockSpec((MO, MV), lambda i: (0, 0)),
            pl.BlockSpec((MV, H1), lambda i: (0, 0)),
            pl.BlockSpec((1, H1), lambda i: (0, 0)),
        ],
        out_specs=pl.BlockSpec((be, D), lambda i: (i, 0)),
        out_shape=jax.ShapeDtypeStruct((e_total, D), _f32),
    )(g, ef8, ones64, wct, wd, w2_msg, w1_mov, wm2b, c64)


def _make_scatter(offsets):
    nslices = len(offsets)
    epw = ES // NW
    nchunk = epw // CH
    assert epw % CH == 0

    def body(*refs):
        dst_hbm = refs[0]
        ms = refs[1:1 + nslices]
        out_hbm = refs[1 + nslices]
        (didx0, rows0, didx1, rows1, zbuf, acc, semr0, semr1) = \
            refs[2 + nslices:]
        cid = lax.axis_index("c")
        sid = lax.axis_index("s")
        wid = sid * NC + cid

        def zrow(r, carry):
            for cc in range(DA // L):
                zbuf[r, pl.ds(cc * L, L)] = jnp.zeros((L,), _f32)
            return carry

        lax.fori_loop(0, ZB, zrow, 0)

        def zcopy(k, carry):
            pltpu.sync_copy(zbuf, acc.at[pl.ds(sid * NPT + k * ZB, ZB)])
            return carry

        lax.fori_loop(0, NPT // ZB, zcopy, 0)
        plsc.subcore_barrier()

        base0 = wid * epw
        didx = (didx0, didx1)
        rows = (rows0, rows1)
        semr = (semr0, semr1)

        for s in range(nslices):
            m_hbm = ms[s]
            goff = offsets[s]

            def fire(c, bufi):
                base = base0 + c * CH
                pltpu.sync_copy(dst_hbm.at[pl.ds(goff + base, CH)],
                                didx[bufi])
                pltpu.async_copy(m_hbm.at[pl.ds(base, CH)], rows[bufi],
                                 semr[bufi])

            def finish(c, bufi):
                base = base0 + c * CH
                pltpu.make_async_copy(m_hbm.at[pl.ds(base, CH)], rows[bufi],
                                      semr[bufi]).wait()
                pltpu.sync_copy(rows[bufi], acc.at[didx[bufi]], add=True)

            _pipeline(nchunk, fire, finish)

        plsc.subcore_barrier()
        pltpu.sync_copy(acc.at[pl.ds(sid * NPT, NPT)],
                        out_hbm.at[cid, pl.ds(sid * NPT, NPT)])

    mesh = plsc.VectorSubcoreMesh(core_axis_name="c", subcore_axis_name="s",
                                  num_cores=NC, num_subcores=NS)

    def call(dst, ms):
        return pl.kernel(
            body,
            out_type=jax.ShapeDtypeStruct((NC, NPA, DA), _f32),
            mesh=mesh,
            scratch_types=[
                pltpu.VMEM((CH,), jnp.int32),
                pltpu.VMEM((CH, D), _f32),
                pltpu.VMEM((CH,), jnp.int32),
                pltpu.VMEM((CH, D), _f32),
                pltpu.VMEM((ZB, DA), _f32),
                pltpu.VMEM_SHARED((NPA, DA), _f32),
                pltpu.SemaphoreType.DMA,
                pltpu.SemaphoreType.DMA,
            ],
        )(dst, *ms)

    return call


def _node_body(nf_ref, coord_ref, pa_ref, pb_ref, w1a_ref, w1b_ref, b1_ref,
               w2_ref, b2_ref, nf_out, co_out):
    nf = nf_ref[...]
    agg = (pa_ref[...][0] + pa_ref[...][1]
           + pb_ref[...][0] + pb_ref[...][1])
    am = agg[:, :MO]
    z = (jnp.dot(nf, w1a_ref[...], preferred_element_type=_f32)
         + jnp.dot(am, w1b_ref[...], preferred_element_type=_f32)
         + b1_ref[...])
    h = z * jax.nn.sigmoid(z)
    nf_out[...] = (nf + jnp.dot(h, w2_ref[...], preferred_element_type=_f32)
                   + b2_ref[...])
    co_out[...] = coord_ref[...] + agg[:, H1:H1 + 3]


def _node(node_feat, coord, parts_a, parts_b, w1a, w1b, b1_row, node_w2,
          b2_row):
    blk = 1000
    grid = N // blk
    return pl.pallas_call(
        _node_body,
        grid=(grid,),
        in_specs=[
            pl.BlockSpec((blk, FN), lambda i: (i, 0)),
            pl.BlockSpec((blk, 3), lambda i: (i, 0)),
            pl.BlockSpec((2, blk, DA), lambda i: (0, i, 0)),
            pl.BlockSpec((2, blk, DA), lambda i: (0, i, 0)),
            pl.BlockSpec((FN, FN), lambda i: (0, 0)),
            pl.BlockSpec((MO, FN), lambda i: (0, 0)),
            pl.BlockSpec((1, FN), lambda i: (0, 0)),
            pl.BlockSpec((FN, FN), lambda i: (0, 0)),
            pl.BlockSpec((1, FN), lambda i: (0, 0)),
        ],
        out_specs=[
            pl.BlockSpec((blk, FN), lambda i: (i, 0)),
            pl.BlockSpec((blk, 3), lambda i: (i, 0)),
        ],
        out_shape=[
            jax.ShapeDtypeStruct((N, FN), _f32),
            jax.ShapeDtypeStruct((N, 3), _f32),
        ],
    )(node_feat, coord, parts_a, parts_b, w1a, w1b, b1_row, node_w2, b2_row)


def kernel(node_feat, coord, edge_index, edge_feat, w1_msg, w2_msg, w1_mov,
           w2_mov, rbf_centers, node_w1, node_b1, node_w2, node_b2):
    wsrc = w1_msg[:FN]
    wdst = w1_msg[FN:2 * FN]
    wc = w1_msg[2 * FN:2 * FN + RBF]
    wd = w1_msg[2 * FN + RBF:]
    ones64 = jnp.ones((H1, H1), _f32)
    wct = jnp.concatenate([wc] * 4, axis=0) * 0.25
    wm2b = jnp.tile(w2_mov, (1, H1))
    c64 = jnp.tile(rbf_centers.reshape(1, RBF), (1, 4))

    src, dst = _split(edge_index)
    a, b = _prep(node_feat, coord, wsrc, wdst)
    ef128 = jnp.pad(edge_feat, ((0, 0), (0, D - FE)))
    wd128 = jnp.pad(wd, ((0, D - FE), (0, 0)))

    gs = [_make_gather(ES, s * ES)(src, dst, a, b) for s in range(NSLICE)]
    ms = [_edge(gs[s], ef128, ones64, wct, wd128, w2_msg, w1_mov, wm2b, c64, s)
          for s in range(NSLICE)]

    parts_a = _make_scatter([0, ES, 2 * ES, 3 * ES])(dst, ms[:4])
    parts_b = _make_scatter([4 * ES])(dst, ms[4:])

    new_feat, new_coord = _node(node_feat, coord, parts_a, parts_b,
                                node_w1[:FN], node_w1[FN:],
                                node_b1.reshape(1, FN), node_w2,
                                node_b2.reshape(1, FN))
    return new_feat, new_coord


# speedup vs baseline: 1.0654x; 1.0654x over previous
import jax
import jax.numpy as jnp
from jax import lax
from jax.experimental import pallas as pl
from jax.experimental.pallas import tpu as pltpu
from jax.experimental.pallas import tpu_sc as plsc

N = 10000
E = 320000
FN = 128
FE = 16
RBF = 16
H1 = 64
MO = 64
MV = 32
GAMMA = 10.0

D = 128
NC, NS, L = 2, 16, 16
NW = NC * NS
CH = 80
NPT = 632
NPA = NPT * NS
DA = 128
ZB = 8

NSLICE = 5
ES = E // NSLICE

_f32 = jnp.float32


def _prep_body(nf_ref, coord_ref, wsrc_ref, wdst_ref, a_ref, b_ref):
    nf = nf_ref[...]
    blk = nf.shape[0]
    c = coord_ref[...]
    cpad = jnp.concatenate([c, jnp.zeros((blk, 16 - 3), _f32)], axis=1)
    zpad = jnp.zeros((blk, D - H1 - 16), _f32)
    p = jnp.dot(nf, wsrc_ref[...], preferred_element_type=_f32)
    q = jnp.dot(nf, wdst_ref[...], preferred_element_type=_f32)
    a_ref[...] = jnp.concatenate([p, -cpad, zpad], axis=1)
    b_ref[...] = jnp.concatenate([q, cpad, zpad], axis=1)


def _prep(node_feat, coord, wsrc, wdst):
    blk = 1000
    grid = N // blk
    return pl.pallas_call(
        _prep_body,
        grid=(grid,),
        in_specs=[
            pl.BlockSpec((blk, FN), lambda i: (i, 0)),
            pl.BlockSpec((blk, 3), lambda i: (i, 0)),
            pl.BlockSpec((FN, H1), lambda i: (0, 0)),
            pl.BlockSpec((FN, H1), lambda i: (0, 0)),
        ],
        out_specs=[
            pl.BlockSpec((blk, D), lambda i: (i, 0)),
            pl.BlockSpec((blk, D), lambda i: (i, 0)),
        ],
        out_shape=[
            jax.ShapeDtypeStruct((N, D), _f32),
            jax.ShapeDtypeStruct((N, D), _f32),
        ],
    )(node_feat, coord, wsrc, wdst)


def _split_body(eix_ref, src_out, dst_out):
    src_out[...] = eix_ref[0, :]
    dst_out[...] = eix_ref[1, :]


def _split(edge_index):
    return pl.pallas_call(
        _split_body,
        out_shape=[
            jax.ShapeDtypeStruct((E,), jnp.int32),
            jax.ShapeDtypeStruct((E,), jnp.int32),
        ],
    )(edge_index)


def _pipeline(nchunk, fire, finish):
    fire(0, 0)
    npairs = (nchunk - 1) // 2

    def pair(j, carry):
        c0 = 2 * j
        fire(c0 + 1, 1)
        finish(c0, 0)
        fire(c0 + 2, 0)
        finish(c0 + 1, 1)
        return carry

    lax.fori_loop(0, npairs, pair, 0)
    if nchunk % 2 == 1:
        finish(nchunk - 1, 0)
    else:
        last = nchunk - 1
        fire(last, 1)
        finish(last - 1, 0)
        finish(last, 1)


def _make_gather(e_total, offset):
    epw = e_total // NW
    nchunk = epw // CH
    assert epw % CH == 0 and (epw % 8) == 0

    assert nchunk % 2 == 1 and nchunk >= 5

    def body(src_hbm, dst_hbm, a_hbm, b_hbm, g_hbm,
             sidx0, didx0, rowsa0, rowsb0, wbuf0,
             sidx1, didx1, rowsa1, rowsb1, wbuf1,
             sema0, semb0, semw0, sema1, semb1, semw1):
        wid = lax.axis_index("s") * NC + lax.axis_index("c")
        base0 = offset + wid * epw
        sidx = (sidx0, sidx1)
        didx = (didx0, didx1)
        rowsa = (rowsa0, rowsa1)
        rowsb = (rowsb0, rowsb1)
        wbuf = (wbuf0, wbuf1)
        sema = (sema0, sema1)
        semb = (semb0, semb1)
        semw = (semw0, semw1)

        def fire(c, bufi):
            base = base0 + c * CH
            pltpu.sync_copy(src_hbm.at[pl.ds(base, CH)], sidx[bufi])
            pltpu.sync_copy(dst_hbm.at[pl.ds(base, CH)], didx[bufi])
            pltpu.async_copy(a_hbm.at[sidx[bufi]], rowsa[bufi], sema[bufi])
            pltpu.async_copy(b_hbm.at[didx[bufi]], rowsb[bufi], semb[bufi])

        def finish(c, bufi, wb_wait):
            base = wid * epw + c * CH
            pltpu.make_async_copy(a_hbm.at[sidx[bufi]], rowsa[bufi],
                                  sema[bufi]).wait()
            pltpu.make_async_copy(b_hbm.at[didx[bufi]], rowsb[bufi],
                                  semb[bufi]).wait()
            if wb_wait:
                pltpu.make_async_copy(
                    wbuf[bufi], g_hbm.at[pl.ds(base - 2 * CH, CH)],
                    semw[bufi]).wait()
            ra = rowsa[bufi]
            rb = rowsb[bufi]
            w = wbuf[bufi]

            def addrow(r, c2):
                for cc in range(D // L):
                    w[r, pl.ds(cc * L, L)] = (ra[r, pl.ds(cc * L, L)]
                                              + rb[r, pl.ds(cc * L, L)])
                return c2

            lax.fori_loop(0, CH, addrow, 0)
            pltpu.async_copy(w, g_hbm.at[pl.ds(base, CH)], semw[bufi])

        fire(0, 0)
        fire(1, 1)
        finish(0, 0, False)
        fire(2, 0)
        finish(1, 1, False)

        def pair(j, carry):
            c0 = 2 * j
            fire(c0 + 1, 1)
            finish(c0, 0, True)
            fire(c0 + 2, 0)
            finish(c0 + 1, 1, True)
            return carry

        lax.fori_loop(1, (nchunk - 1) // 2, pair, 0)
        finish(nchunk - 1, 0, True)
        pltpu.make_async_copy(
            wbuf[0], g_hbm.at[pl.ds(wid * epw + (nchunk - 1) * CH, CH)],
            semw[0]).wait()
        pltpu.make_async_copy(
            wbuf[1], g_hbm.at[pl.ds(wid * epw + (nchunk - 2) * CH, CH)],
            semw[1]).wait()

    mesh = plsc.VectorSubcoreMesh(core_axis_name="c", subcore_axis_name="s",
                                  num_cores=NC, num_subcores=NS)

    def call(src, dst, a, b):
        return pl.kernel(
            body,
            out_type=jax.ShapeDtypeStruct((e_total, D), _f32),
            mesh=mesh,
            scratch_types=[
                pltpu.VMEM((CH,), jnp.int32),
                pltpu.VMEM((CH,), jnp.int32),
                pltpu.VMEM((CH, D), _f32),
                pltpu.VMEM((CH, D), _f32),
                pltpu.VMEM((CH, D), _f32),
                pltpu.VMEM((CH,), jnp.int32),
                pltpu.VMEM((CH,), jnp.int32),
                pltpu.VMEM((CH, D), _f32),
                pltpu.VMEM((CH, D), _f32),
                pltpu.VMEM((CH, D), _f32),
                pltpu.SemaphoreType.DMA,
                pltpu.SemaphoreType.DMA,
                pltpu.SemaphoreType.DMA,
                pltpu.SemaphoreType.DMA,
                pltpu.SemaphoreType.DMA,
                pltpu.SemaphoreType.DMA,
            ],
        )(src, dst, a, b)

    return call


def _edge_body(g_ref, ef_ref, ones_ref, wct_ref, wd_ref, w2_ref, wm1_ref,
               wm2b_ref, c64_ref, out_ref):
    g = g_ref[...]
    pre = g[:, :H1]
    gt = g[:, H1:]
    ef = ef_ref[...]
    d2 = jnp.dot(gt * gt, ones_ref[...], preferred_element_type=_f32)
    dist = jnp.sqrt(d2 + 1e-12)
    diff = dist - c64_ref[...]
    rbfx = jnp.exp(-GAMMA * diff * diff)
    z = (pre
         + jnp.dot(rbfx, wct_ref[...], preferred_element_type=_f32)
         + jnp.dot(ef, wd_ref[...], preferred_element_type=_f32))
    h = z * jax.nn.sigmoid(z)
    m = jnp.dot(h, w2_ref[...], preferred_element_type=_f32)
    m = m * jax.nn.sigmoid(m)
    mh = jnp.dot(m, wm1_ref[...], preferred_element_type=_f32)
    mh = mh * jax.nn.sigmoid(mh)
    scalb = jnp.dot(mh, wm2b_ref[...], preferred_element_type=_f32)
    out_ref[...] = jnp.concatenate([m, gt * scalb], axis=1)


def _edge(g, ef8, ones64, wct, wd, w2_msg, w1_mov, wm2b, c64, sl):
    e_total = g.shape[0]
    be = 2560
    grid = e_total // be
    boff = sl * (ES // be)
    return pl.pallas_call(
        _edge_body,
        grid=(grid,),
        in_specs=[
            pl.BlockSpec((be, D), lambda i: (i, 0)),
            pl.BlockSpec((be, FE), lambda i, boff=boff: (i + boff, 0)),
            pl.BlockSpec((H1, H1), lambda i: (0, 0)),
            pl.BlockSpec((H1, H1), lambda i: (0, 0)),
            pl.BlockSpec((FE, H1), lambda i: (0, 0)),
            pl.BlockSpec((H1, MO), lambda i: (0, 0)),
            pl.BlockSpec((MO, MV), lambda i: (0, 0)),
            pl.BlockSpec((MV, H1), lambda i: (0, 0)),
            pl.BlockSpec((1, H1), lambda i: (0, 0)),
        ],
        out_specs=pl.BlockSpec((be, D), lambda i: (i, 0)),
        out_shape=jax.ShapeDtypeStruct((e_total, D), _f32),
    )(g, ef8, ones64, wct, wd, w2_msg, w1_mov, wm2b, c64)


def _make_scatter(offsets):
    nslices = len(offsets)
    epw = ES // NW
    nchunk = epw // CH
    assert epw % CH == 0

    def body(*refs):
        dst_hbm = refs[0]
        ms = refs[1:1 + nslices]
        out_hbm = refs[1 + nslices]
        (didx0, rows0, didx1, rows1, zbuf, acc, semr0, semr1) = \
            refs[2 + nslices:]
        cid = lax.axis_index("c")
        sid = lax.axis_index("s")
        wid = sid * NC + cid

        def zrow(r, carry):
            for cc in range(DA // L):
                zbuf[r, pl.ds(cc * L, L)] = jnp.zeros((L,), _f32)
            return carry

        lax.fori_loop(0, ZB, zrow, 0)

        def zcopy(k, carry):
            pltpu.sync_copy(zbuf, acc.at[pl.ds(sid * NPT + k * ZB, ZB)])
            return carry

        lax.fori_loop(0, NPT // ZB, zcopy, 0)
        plsc.subcore_barrier()

        base0 = wid * epw
        didx = (didx0, didx1)
        rows = (rows0, rows1)
        semr = (semr0, semr1)

        for s in range(nslices):
            m_hbm = ms[s]
            goff = offsets[s]

            def fire(c, bufi):
                base = base0 + c * CH
                pltpu.sync_copy(dst_hbm.at[pl.ds(goff + base, CH)],
                                didx[bufi])
                pltpu.async_copy(m_hbm.at[pl.ds(base, CH)], rows[bufi],
                                 semr[bufi])

            def finish(c, bufi):
                base = base0 + c * CH
                pltpu.make_async_copy(m_hbm.at[pl.ds(base, CH)], rows[bufi],
                                      semr[bufi]).wait()
                pltpu.sync_copy(rows[bufi], acc.at[didx[bufi]], add=True)

            _pipeline(nchunk, fire, finish)

        plsc.subcore_barrier()
        pltpu.sync_copy(acc.at[pl.ds(sid * NPT, NPT)],
                        out_hbm.at[cid, pl.ds(sid * NPT, NPT)])

    mesh = plsc.VectorSubcoreMesh(core_axis_name="c", subcore_axis_name="s",
                                  num_cores=NC, num_subcores=NS)

    def call(dst, ms):
        return pl.kernel(
            body,
            out_type=jax.ShapeDtypeStruct((NC, NPA, DA), _f32),
            mesh=mesh,
            scratch_types=[
                pltpu.VMEM((CH,), jnp.int32),
                pltpu.VMEM((CH, D), _f32),
                pltpu.VMEM((CH,), jnp.int32),
                pltpu.VMEM((CH, D), _f32),
                pltpu.VMEM((ZB, DA), _f32),
                pltpu.VMEM_SHARED((NPA, DA), _f32),
                pltpu.SemaphoreType.DMA,
                pltpu.SemaphoreType.DMA,
            ],
        )(dst, *ms)

    return call


def _node_body(nf_ref, coord_ref, pa_ref, pb_ref, w1a_ref, w1b_ref, b1_ref,
               w2_ref, b2_ref, nf_out, co_out):
    nf = nf_ref[...]
    agg = (pa_ref[...][0] + pa_ref[...][1]
           + pb_ref[...][0] + pb_ref[...][1])
    am = agg[:, :MO]
    z = (jnp.dot(nf, w1a_ref[...], preferred_element_type=_f32)
         + jnp.dot(am, w1b_ref[...], preferred_element_type=_f32)
         + b1_ref[...])
    h = z * jax.nn.sigmoid(z)
    nf_out[...] = (nf + jnp.dot(h, w2_ref[...], preferred_element_type=_f32)
                   + b2_ref[...])
    co_out[...] = coord_ref[...] + agg[:, H1:H1 + 3]


def _node(node_feat, coord, parts_a, parts_b, w1a, w1b, b1_row, node_w2,
          b2_row):
    blk = 1000
    grid = N // blk
    return pl.pallas_call(
        _node_body,
        grid=(grid,),
        in_specs=[
            pl.BlockSpec((blk, FN), lambda i: (i, 0)),
            pl.BlockSpec((blk, 3), lambda i: (i, 0)),
            pl.BlockSpec((2, blk, DA), lambda i: (0, i, 0)),
            pl.BlockSpec((2, blk, DA), lambda i: (0, i, 0)),
            pl.BlockSpec((FN, FN), lambda i: (0, 0)),
            pl.BlockSpec((MO, FN), lambda i: (0, 0)),
            pl.BlockSpec((1, FN), lambda i: (0, 0)),
            pl.BlockSpec((FN, FN), lambda i: (0, 0)),
            pl.BlockSpec((1, FN), lambda i: (0, 0)),
        ],
        out_specs=[
            pl.BlockSpec((blk, FN), lambda i: (i, 0)),
            pl.BlockSpec((blk, 3), lambda i: (i, 0)),
        ],
        out_shape=[
            jax.ShapeDtypeStruct((N, FN), _f32),
            jax.ShapeDtypeStruct((N, 3), _f32),
        ],
    )(node_feat, coord, parts_a, parts_b, w1a, w1b, b1_row, node_w2, b2_row)


def kernel(node_feat, coord, edge_index, edge_feat, w1_msg, w2_msg, w1_mov,
           w2_mov, rbf_centers, node_w1, node_b1, node_w2, node_b2):
    wsrc = w1_msg[:FN]
    wdst = w1_msg[FN:2 * FN]
    wc = w1_msg[2 * FN:2 * FN + RBF]
    wd = w1_msg[2 * FN + RBF:]
    ones64 = jnp.ones((H1, H1), _f32)
    wct = jnp.concatenate([wc] * 4, axis=0) * 0.25
    wm2b = jnp.tile(w2_mov, (1, H1))
    c64 = jnp.tile(rbf_centers.reshape(1, RBF), (1, 4))

    src, dst = _split(edge_index)
    a, b = _prep(node_feat, coord, wsrc, wdst)
    gs = [_make_gather(ES, s * ES)(src, dst, a, b) for s in range(NSLICE)]
    ms = [_edge(gs[s], edge_feat, ones64, wct, wd, w2_msg, w1_mov, wm2b, c64, s)
          for s in range(NSLICE)]

    parts_a = _make_scatter([0, ES, 2 * ES, 3 * ES])(dst, ms[:4])
    parts_b = _make_scatter([4 * ES])(dst, ms[4:])

    new_feat, new_coord = _node(node_feat, coord, parts_a, parts_b,
                                node_w1[:FN], node_w1[FN:],
                                node_b1.reshape(1, FN), node_w2,
                                node_b2.reshape(1, FN))
    return new_feat, new_coord


# speedup vs baseline: 1.1807x; 1.1082x over previous
import jax
import jax.numpy as jnp
from jax import lax
from jax.experimental import pallas as pl
from jax.experimental.pallas import tpu as pltpu
from jax.experimental.pallas import tpu_sc as plsc

N = 10000
E = 320000
FN = 128
FE = 16
RBF = 16
H1 = 64
MO = 64
MV = 32
GAMMA = 10.0

D = 128
NC, NS, L = 2, 16, 16
NW = NC * NS
CH = 80
NPT = 632
NPA = NPT * NS
DA = 128
ZB = 8

NSLICE = 5
ES = E // NSLICE

_f32 = jnp.float32


def _prep_body(nf_ref, coord_ref, wsrc_ref, wdst_ref, a_ref, b_ref):
    nf = nf_ref[...]
    blk = nf.shape[0]
    c = coord_ref[...]
    cpad = jnp.concatenate([c, jnp.zeros((blk, 16 - 3), _f32)], axis=1)
    zpad = jnp.zeros((blk, D - H1 - 16), _f32)
    p = jnp.dot(nf, wsrc_ref[...], preferred_element_type=_f32)
    q = jnp.dot(nf, wdst_ref[...], preferred_element_type=_f32)
    a_ref[...] = jnp.concatenate([p, -cpad, zpad], axis=1)
    b_ref[...] = jnp.concatenate([q, cpad, zpad], axis=1)


def _prep(node_feat, coord, wsrc, wdst):
    blk = 1000
    grid = N // blk
    return pl.pallas_call(
        _prep_body,
        grid=(grid,),
        in_specs=[
            pl.BlockSpec((blk, FN), lambda i: (i, 0)),
            pl.BlockSpec((blk, 3), lambda i: (i, 0)),
            pl.BlockSpec((FN, H1), lambda i: (0, 0)),
            pl.BlockSpec((FN, H1), lambda i: (0, 0)),
        ],
        out_specs=[
            pl.BlockSpec((blk, D), lambda i: (i, 0)),
            pl.BlockSpec((blk, D), lambda i: (i, 0)),
        ],
        out_shape=[
            jax.ShapeDtypeStruct((N, D), _f32),
            jax.ShapeDtypeStruct((N, D), _f32),
        ],
    )(node_feat, coord, wsrc, wdst)


def _split_body(eix_ref, src_out, dst_out):
    src_out[...] = eix_ref[0, :]
    dst_out[...] = eix_ref[1, :]


def _split(edge_index):
    return pl.pallas_call(
        _split_body,
        out_shape=[
            jax.ShapeDtypeStruct((E,), jnp.int32),
            jax.ShapeDtypeStruct((E,), jnp.int32),
        ],
    )(edge_index)


def _pipeline(nchunk, fire, finish):
    fire(0, 0)
    npairs = (nchunk - 1) // 2

    def pair(j, carry):
        c0 = 2 * j
        fire(c0 + 1, 1)
        finish(c0, 0)
        fire(c0 + 2, 0)
        finish(c0 + 1, 1)
        return carry

    lax.fori_loop(0, npairs, pair, 0)
    if nchunk % 2 == 1:
        finish(nchunk - 1, 0)
    else:
        last = nchunk - 1
        fire(last, 1)
        finish(last - 1, 0)
        finish(last, 1)


def _make_gather(e_total, offset):
    epw = e_total // NW
    nchunk = epw // CH
    assert epw % CH == 0 and (epw % 8) == 0

    assert nchunk % 2 == 1 and nchunk >= 5

    def body(src_hbm, dst_hbm, a_hbm, b_hbm, g_hbm,
             sidx0, didx0, rowsa0, rowsb0, wbuf0,
             sidx1, didx1, rowsa1, rowsb1, wbuf1,
             sema0, semb0, semw0, sema1, semb1, semw1):
        wid = lax.axis_index("s") * NC + lax.axis_index("c")
        base0 = offset + wid * epw
        sidx = (sidx0, sidx1)
        didx = (didx0, didx1)
        rowsa = (rowsa0, rowsa1)
        rowsb = (rowsb0, rowsb1)
        wbuf = (wbuf0, wbuf1)
        sema = (sema0, sema1)
        semb = (semb0, semb1)
        semw = (semw0, semw1)

        def fire(c, bufi):
            base = base0 + c * CH
            pltpu.sync_copy(src_hbm.at[pl.ds(base, CH)], sidx[bufi])
            pltpu.sync_copy(dst_hbm.at[pl.ds(base, CH)], didx[bufi])
            pltpu.async_copy(a_hbm.at[sidx[bufi]], rowsa[bufi], sema[bufi])
            pltpu.async_copy(b_hbm.at[didx[bufi]], rowsb[bufi], semb[bufi])

        def finish(c, bufi, wb_wait):
            base = wid * epw + c * CH
            pltpu.make_async_copy(a_hbm.at[sidx[bufi]], rowsa[bufi],
                                  sema[bufi]).wait()
            pltpu.make_async_copy(b_hbm.at[didx[bufi]], rowsb[bufi],
                                  semb[bufi]).wait()
            if wb_wait:
                pltpu.make_async_copy(
                    wbuf[bufi], g_hbm.at[pl.ds(base - 2 * CH, CH)],
                    semw[bufi]).wait()
            ra = rowsa[bufi]
            rb = rowsb[bufi]
            w = wbuf[bufi]

            def addrow(r, c2):
                for cc in range(D // L):
                    w[r, pl.ds(cc * L, L)] = (ra[r, pl.ds(cc * L, L)]
                                              + rb[r, pl.ds(cc * L, L)])
                return c2

            lax.fori_loop(0, CH, addrow, 0)
            pltpu.async_copy(w, g_hbm.at[pl.ds(base, CH)], semw[bufi])

        fire(0, 0)
        fire(1, 1)
        finish(0, 0, False)
        fire(2, 0)
        finish(1, 1, False)

        def pair(j, carry):
            c0 = 2 * j
            fire(c0 + 1, 1)
            finish(c0, 0, True)
            fire(c0 + 2, 0)
            finish(c0 + 1, 1, True)
            return carry

        lax.fori_loop(1, (nchunk - 1) // 2, pair, 0)
        finish(nchunk - 1, 0, True)
        pltpu.make_async_copy(
            wbuf[0], g_hbm.at[pl.ds(wid * epw + (nchunk - 1) * CH, CH)],
            semw[0]).wait()
        pltpu.make_async_copy(
            wbuf[1], g_hbm.at[pl.ds(wid * epw + (nchunk - 2) * CH, CH)],
            semw[1]).wait()

    mesh = plsc.VectorSubcoreMesh(core_axis_name="c", subcore_axis_name="s",
                                  num_cores=NC, num_subcores=NS)

    def call(src, dst, a, b):
        return pl.kernel(
            body,
            out_type=jax.ShapeDtypeStruct((e_total, D), _f32),
            mesh=mesh,
            scratch_types=[
                pltpu.VMEM((CH,), jnp.int32),
                pltpu.VMEM((CH,), jnp.int32),
                pltpu.VMEM((CH, D), _f32),
                pltpu.VMEM((CH, D), _f32),
                pltpu.VMEM((CH, D), _f32),
                pltpu.VMEM((CH,), jnp.int32),
                pltpu.VMEM((CH,), jnp.int32),
                pltpu.VMEM((CH, D), _f32),
                pltpu.VMEM((CH, D), _f32),
                pltpu.VMEM((CH, D), _f32),
                pltpu.SemaphoreType.DMA,
                pltpu.SemaphoreType.DMA,
                pltpu.SemaphoreType.DMA,
                pltpu.SemaphoreType.DMA,
                pltpu.SemaphoreType.DMA,
                pltpu.SemaphoreType.DMA,
            ],
        )(src, dst, a, b)

    return call


def _edge_body(g_ref, ef_ref, ones_ref, wct_ref, wd_ref, w2_ref, wm1_ref,
               wm2b_ref, c64_ref, out_ref):
    g = g_ref[...]
    pre = g[:, :H1]
    gt = g[:, H1:]
    ef = ef_ref[...]
    d2 = jnp.dot(gt * gt, ones_ref[...], preferred_element_type=_f32)
    dist = jnp.sqrt(d2 + 1e-12)
    diff = dist - c64_ref[...]
    rbfx = jnp.exp(-GAMMA * diff * diff)
    z = (pre
         + jnp.dot(rbfx, wct_ref[...], preferred_element_type=_f32)
         + jnp.dot(ef, wd_ref[...], preferred_element_type=_f32))
    h = z * jax.nn.sigmoid(z)
    m = jnp.dot(h, w2_ref[...], preferred_element_type=_f32)
    m = m * jax.nn.sigmoid(m)
    mh = jnp.dot(m, wm1_ref[...], preferred_element_type=_f32)
    mh = mh * jax.nn.sigmoid(mh)
    scalb = jnp.dot(mh, wm2b_ref[...], preferred_element_type=_f32)
    out_ref[...] = jnp.concatenate([m, gt * scalb], axis=1)


def _edge(g, ef8, ones64, wct, wd, w2_msg, w1_mov, wm2b, c64, sl):
    e_total = g.shape[0]
    be = 6400
    grid = e_total // be
    boff = sl * (ES // be)
    return pl.pallas_call(
        _edge_body,
        grid=(grid,),
        in_specs=[
            pl.BlockSpec((be, D), lambda i: (i, 0)),
            pl.BlockSpec((be, FE), lambda i, boff=boff: (i + boff, 0)),
            pl.BlockSpec((H1, H1), lambda i: (0, 0)),
            pl.BlockSpec((H1, H1), lambda i: (0, 0)),
            pl.BlockSpec((FE, H1), lambda i: (0, 0)),
            pl.BlockSpec((H1, MO), lambda i: (0, 0)),
            pl.BlockSpec((MO, MV), lambda i: (0, 0)),
            pl.BlockSpec((MV, H1), lambda i: (0, 0)),
            pl.BlockSpec((1, H1), lambda i: (0, 0)),
        ],
        out_specs=pl.BlockSpec((be, D), lambda i: (i, 0)),
        out_shape=jax.ShapeDtypeStruct((e_total, D), _f32),
    )(g, ef8, ones64, wct, wd, w2_msg, w1_mov, wm2b, c64)


def _make_scatter(offsets):
    nslices = len(offsets)
    epw = ES // NW
    nchunk = epw // CH
    assert epw % CH == 0

    def body(*refs):
        dst_hbm = refs[0]
        ms = refs[1:1 + nslices]
        out_hbm = refs[1 + nslices]
        (didx0, rows0, didx1, rows1, zbuf, acc, semr0, semr1) = \
            refs[2 + nslices:]
        cid = lax.axis_index("c")
        sid = lax.axis_index("s")
        wid = sid * NC + cid

        def zrow(r, carry):
            for cc in range(DA // L):
                zbuf[r, pl.ds(cc * L, L)] = jnp.zeros((L,), _f32)
            return carry

        lax.fori_loop(0, ZB, zrow, 0)

        def zcopy(k, carry):
            pltpu.sync_copy(zbuf, acc.at[pl.ds(sid * NPT + k * ZB, ZB)])
            return carry

        lax.fori_loop(0, NPT // ZB, zcopy, 0)
        plsc.subcore_barrier()

        base0 = wid * epw
        didx = (didx0, didx1)
        rows = (rows0, rows1)
        semr = (semr0, semr1)

        for s in range(nslices):
            m_hbm = ms[s]
            goff = offsets[s]

            def fire(c, bufi):
                base = base0 + c * CH
                pltpu.sync_copy(dst_hbm.at[pl.ds(goff + base, CH)],
                                didx[bufi])
                pltpu.async_copy(m_hbm.at[pl.ds(base, CH)], rows[bufi],
                                 semr[bufi])

            def finish(c, bufi):
                base = base0 + c * CH
                pltpu.make_async_copy(m_hbm.at[pl.ds(base, CH)], rows[bufi],
                                      semr[bufi]).wait()
                pltpu.sync_copy(rows[bufi], acc.at[didx[bufi]], add=True)

            _pipeline(nchunk, fire, finish)

        plsc.subcore_barrier()
        pltpu.sync_copy(acc.at[pl.ds(sid * NPT, NPT)],
                        out_hbm.at[cid, pl.ds(sid * NPT, NPT)])

    mesh = plsc.VectorSubcoreMesh(core_axis_name="c", subcore_axis_name="s",
                                  num_cores=NC, num_subcores=NS)

    def call(dst, ms):
        return pl.kernel(
            body,
            out_type=jax.ShapeDtypeStruct((NC, NPA, DA), _f32),
            mesh=mesh,
            scratch_types=[
                pltpu.VMEM((CH,), jnp.int32),
                pltpu.VMEM((CH, D), _f32),
                pltpu.VMEM((CH,), jnp.int32),
                pltpu.VMEM((CH, D), _f32),
                pltpu.VMEM((ZB, DA), _f32),
                pltpu.VMEM_SHARED((NPA, DA), _f32),
                pltpu.SemaphoreType.DMA,
                pltpu.SemaphoreType.DMA,
            ],
        )(dst, *ms)

    return call


def _node_body(nf_ref, coord_ref, pa_ref, pb_ref, w1a_ref, w1b_ref, b1_ref,
               w2_ref, b2_ref, nf_out, co_out):
    nf = nf_ref[...]
    agg = (pa_ref[...][0] + pa_ref[...][1]
           + pb_ref[...][0] + pb_ref[...][1])
    am = agg[:, :MO]
    z = (jnp.dot(nf, w1a_ref[...], preferred_element_type=_f32)
         + jnp.dot(am, w1b_ref[...], preferred_element_type=_f32)
         + b1_ref[...])
    h = z * jax.nn.sigmoid(z)
    nf_out[...] = (nf + jnp.dot(h, w2_ref[...], preferred_element_type=_f32)
                   + b2_ref[...])
    co_out[...] = coord_ref[...] + agg[:, H1:H1 + 3]


def _node(node_feat, coord, parts_a, parts_b, w1a, w1b, b1_row, node_w2,
          b2_row):
    blk = 1000
    grid = N // blk
    return pl.pallas_call(
        _node_body,
        grid=(grid,),
        in_specs=[
            pl.BlockSpec((blk, FN), lambda i: (i, 0)),
            pl.BlockSpec((blk, 3), lambda i: (i, 0)),
            pl.BlockSpec((2, blk, DA), lambda i: (0, i, 0)),
            pl.BlockSpec((2, blk, DA), lambda i: (0, i, 0)),
            pl.BlockSpec((FN, FN), lambda i: (0, 0)),
            pl.BlockSpec((MO, FN), lambda i: (0, 0)),
            pl.BlockSpec((1, FN), lambda i: (0, 0)),
            pl.BlockSpec((FN, FN), lambda i: (0, 0)),
            pl.BlockSpec((1, FN), lambda i: (0, 0)),
        ],
        out_specs=[
            pl.BlockSpec((blk, FN), lambda i: (i, 0)),
            pl.BlockSpec((blk, 3), lambda i: (i, 0)),
        ],
        out_shape=[
            jax.ShapeDtypeStruct((N, FN), _f32),
            jax.ShapeDtypeStruct((N, 3), _f32),
        ],
    )(node_feat, coord, parts_a, parts_b, w1a, w1b, b1_row, node_w2, b2_row)


def kernel(node_feat, coord, edge_index, edge_feat, w1_msg, w2_msg, w1_mov,
           w2_mov, rbf_centers, node_w1, node_b1, node_w2, node_b2):
    wsrc = w1_msg[:FN]
    wdst = w1_msg[FN:2 * FN]
    wc = w1_msg[2 * FN:2 * FN + RBF]
    wd = w1_msg[2 * FN + RBF:]
    ones64 = jnp.ones((H1, H1), _f32)
    wct = jnp.concatenate([wc] * 4, axis=0) * 0.25
    wm2b = jnp.tile(w2_mov, (1, H1))
    c64 = jnp.tile(rbf_centers.reshape(1, RBF), (1, 4))

    src, dst = _split(edge_index)
    a, b = _prep(node_feat, coord, wsrc, wdst)
    gs = [_make_gather(ES, s * ES)(src, dst, a, b) for s in range(NSLICE)]
    ef_bf = edge_feat.astype(jnp.bfloat16)
    wd_bf = wd.astype(jnp.bfloat16)
    ms = [_edge(gs[s], ef_bf, ones64, wct, wd_bf, w2_msg, w1_mov, wm2b, c64, s)
          for s in range(NSLICE)]

    parts_a = _make_scatter([0, ES, 2 * ES, 3 * ES])(dst, ms[:4])
    parts_b = _make_scatter([4 * ES])(dst, ms[4:])

    new_feat, new_coord = _node(node_feat, coord, parts_a, parts_b,
                                node_w1[:FN], node_w1[FN:],
                                node_b1.reshape(1, FN), node_w2,
                                node_b2.reshape(1, FN))
    return new_feat, new_coord
